# baseline (speedup 1.0000x reference)
import functools

import jax
import jax.numpy as jnp
from jax.experimental import pallas as pl
from jax.experimental.pallas import tpu as pltpu


def _fused_body(W, x_ref, w2_ref, wdw_ref, bias_ref, o_ref):
    Q = x_ref.shape[1]
    Cout = wdw_ref.shape[1]
    Wh = W // 2

    y2 = jnp.dot(x_ref[0], w2_ref[...],
                 preferred_element_type=jnp.float32)
    ylo = y2[:, :Cout]
    yhi = y2[:, Cout:]

    q = jax.lax.broadcasted_iota(jnp.int32, (Q, Cout), 0)
    q_in_row = q & (Wh - 1)

    llo = pltpu.roll(yhi, shift=1, axis=0)
    llo = jnp.where(q_in_row > 0, llo, 0.0)
    lhi = ylo
    rlo = yhi
    rhi = pltpu.roll(ylo, shift=Q - 1, axis=0)
    rhi = jnp.where(q_in_row < Wh - 1, rhi, 0.0)

    wdw = wdw_ref[...]

    def trow(di, l, y, r):
        return (l * wdw[3 * di + 0:3 * di + 1, :]
                + y * wdw[3 * di + 1:3 * di + 2, :]
                + r * wdw[3 * di + 2:3 * di + 3, :])

    bias = bias_ref[...]
    for half, (l, y, r) in enumerate(((llo, ylo, rlo), (lhi, yhi, rhi))):
        tm = pltpu.roll(trow(0, l, y, r), shift=Wh, axis=0)
        tp = pltpu.roll(trow(2, l, y, r), shift=Q - Wh, axis=0)
        acc = (trow(1, l, y, r)
               + jnp.where(q >= Wh, tm, 0.0)
               + jnp.where(q < Q - Wh, tp, 0.0))
        o_ref[0, :, half * Cout:(half + 1) * Cout] = (
            jnp.maximum(acc + bias, 0.0))


def kernel(x_nchw, w1, wdw, gamma, beta, run_mean, run_var):
    eps = 1e-5
    N, Cin, H, W = x_nchw.shape
    Cout = w1.shape[0]
    HW = H * W
    Q = HW // 2
    f32 = jnp.float32

    inv = (gamma.astype(f32) / jnp.sqrt(run_var.astype(f32) + eps))
    bias = (beta.astype(f32) - run_mean.astype(f32) * inv)

    xp = jax.lax.reshape(x_nchw, (N, Q, 2 * Cin), dimensions=(0, 2, 3, 1))

    w1t = jnp.transpose(w1.astype(f32), (1, 0))
    w2 = jnp.zeros((2 * Cin, 2 * Cout), f32)
    w2 = w2.at[:Cin, :Cout].set(w1t).at[Cin:, Cout:].set(w1t)
    wdw_k = (wdw.astype(f32) * inv[:, None, None]).reshape(Cout, 9)
    wdw_k = jnp.transpose(wdw_k, (1, 0))
    bias_k = bias[None, :]

    flops = 2 * N * Q * (2 * Cin) * (2 * Cout) + 19 * N * HW * Cout
    bytes_accessed = 4 * (xp.size + w2.size + wdw_k.size + bias_k.size
                          + N * Cout * HW)

    out = pl.pallas_call(
        functools.partial(_fused_body, W),
        out_shape=jax.ShapeDtypeStruct((N, Q, 2 * Cout), f32),
        grid=(N,),
        in_specs=[
            pl.BlockSpec((1, Q, 2 * Cin), lambda n: (n, 0, 0)),
            pl.BlockSpec((2 * Cin, 2 * Cout), lambda n: (0, 0)),
            pl.BlockSpec((9, Cout), lambda n: (0, 0)),
            pl.BlockSpec((1, Cout), lambda n: (0, 0)),
        ],
        out_specs=pl.BlockSpec((1, Q, 2 * Cout), lambda n: (n, 0, 0)),
        compiler_params=pltpu.CompilerParams(
            dimension_semantics=("parallel",),
            vmem_limit_bytes=100 * 1024 * 1024,
        ),
        cost_estimate=pl.CostEstimate(
            flops=flops, transcendentals=0, bytes_accessed=bytes_accessed),
    )(xp, w2, wdw_k, bias_k)

    return jnp.transpose(out.reshape(N, H, W, Cout), (0, 3, 1, 2))


# speedup vs baseline: 1.9489x; 1.9489x over previous
import functools

import jax
import jax.numpy as jnp
from jax.experimental import pallas as pl
from jax.experimental.pallas import tpu as pltpu


def _fused_body(W, x_ref, w1t_ref, wdw_ref, bias_ref, o_ref):
    HW = x_ref.shape[1]
    Cout = w1t_ref.shape[1]

    y = jnp.dot(x_ref[0], w1t_ref[...],
                preferred_element_type=jnp.float32)

    row = jax.lax.broadcasted_iota(jnp.int32, (HW, Cout), 0)
    w_in_row = row & (W - 1)
    l = pltpu.roll(y, shift=1, axis=0)
    l = jnp.where(w_in_row > 0, l, 0.0)
    r = pltpu.roll(y, shift=HW - 1, axis=0)
    r = jnp.where(w_in_row < W - 1, r, 0.0)

    wdw = wdw_ref[...]

    def trow(di):
        return (l * wdw[3 * di + 0:3 * di + 1, :]
                + y * wdw[3 * di + 1:3 * di + 2, :]
                + r * wdw[3 * di + 2:3 * di + 3, :])

    tm = pltpu.roll(trow(0), shift=W, axis=0)
    tp = pltpu.roll(trow(2), shift=HW - W, axis=0)
    acc = (trow(1)
           + jnp.where(row >= W, tm, 0.0)
           + jnp.where(row < HW - W, tp, 0.0))

    o_ref[0] = jnp.maximum(acc + bias_ref[...], 0.0)


def kernel(x_nchw, w1, wdw, gamma, beta, run_mean, run_var):
    eps = 1e-5
    N, Cin, H, W = x_nchw.shape
    Cout = w1.shape[0]
    HW = H * W
    f32 = jnp.float32

    inv = (gamma.astype(f32) / jnp.sqrt(run_var.astype(f32) + eps))
    bias = (beta.astype(f32) - run_mean.astype(f32) * inv)

    x2 = jnp.transpose(x_nchw, (0, 2, 3, 1)).reshape(N, HW, Cin)
    w1t = jnp.transpose(w1.astype(f32), (1, 0))
    wdw_k = (wdw.astype(f32) * inv[:, None, None]).reshape(Cout, 9)
    wdw_k = jnp.transpose(wdw_k, (1, 0))
    bias_k = bias[None, :]

    flops = 2 * N * HW * Cin * Cout + 19 * N * HW * Cout
    bytes_accessed = 4 * (x2.size + w1t.size + wdw_k.size + bias_k.size
                          + N * Cout * HW)

    out = pl.pallas_call(
        functools.partial(_fused_body, W),
        out_shape=jax.ShapeDtypeStruct((N, HW, Cout), f32),
        grid=(N,),
        in_specs=[
            pl.BlockSpec((1, HW, Cin), lambda n: (n, 0, 0)),
            pl.BlockSpec((Cin, Cout), lambda n: (0, 0)),
            pl.BlockSpec((9, Cout), lambda n: (0, 0)),
            pl.BlockSpec((1, Cout), lambda n: (0, 0)),
        ],
        out_specs=pl.BlockSpec((1, HW, Cout), lambda n: (n, 0, 0)),
        compiler_params=pltpu.CompilerParams(
            dimension_semantics=("parallel",),
            vmem_limit_bytes=100 * 1024 * 1024,
        ),
        cost_estimate=pl.CostEstimate(
            flops=flops, transcendentals=0, bytes_accessed=bytes_accessed),
    )(x2, w1t, wdw_k, bias_k)

    return jnp.transpose(out.reshape(N, H, W, Cout), (0, 3, 1, 2))
